# baseline (speedup 1.0000x reference)
import jax
import jax.numpy as jnp
from jax.experimental import pallas as pl
from jax.experimental.pallas import tpu as pltpu

_C = 2048
_CTILE = 512
_CH = 256
_NBUF = 4


def _cheb_kernel(
    x0t_ref, w_ref, b_ref, l_hbm, out_ref, ltop_scr, x0_scr, x1_scr, bufs, sems, ltop_sem
):
    f = x0t_ref.shape[0]
    v = x0t_ref.shape[1]
    nstream = (v - _C) // _CH
    total = 2 * nstream
    ncached = _C // _CTILE

    half = _CH // 2

    def chunk_copies(i):
        row = _C + (i % nstream) * _CH
        slot = i % _NBUF
        return [
            pltpu.make_async_copy(
                l_hbm.at[pl.ds(row + h * half, half), :],
                bufs.at[slot, pl.ds(h * half, half), :],
                sems.at[slot, h],
            )
            for h in range(2)
        ]

    def start_chunk(i):
        for c in chunk_copies(i):
            c.start()

    def wait_chunk(i):
        for c in chunk_copies(i):
            c.wait()

    qc = _C // 4
    ltop_copies = [
        pltpu.make_async_copy(
            l_hbm.at[pl.ds(q * qc, qc), :],
            ltop_scr.at[pl.ds(q * qc, qc), :],
            ltop_sem.at[q],
        )
        for q in range(4)
    ]
    for c in ltop_copies:
        c.start()
    for i in range(_NBUF - 1):
        start_chunk(i)

    x0_scr[...] = jnp.transpose(x0t_ref[...], (1, 0))

    def second_pass_tile(l_tile, row, width):
        y = jnp.dot(l_tile, x1_scr[...], preferred_element_type=jnp.float32)
        x0_r = x0_scr[pl.ds(row, width), :]
        x1_r = x1_scr[pl.ds(row, width), :]
        x2_r = 2.0 * y - x0_r
        acc = jnp.dot(x0_r, w_ref[0:f, :], preferred_element_type=jnp.float32)
        acc += jnp.dot(x1_r, w_ref[f : 2 * f, :], preferred_element_type=jnp.float32)
        acc += jnp.dot(x2_r, w_ref[2 * f : 3 * f, :], preferred_element_type=jnp.float32)
        out_ref[:, pl.ds(row, width)] = jnp.transpose(acc, (1, 0)) + b_ref[...]

    for c in ltop_copies:
        c.wait()
    for t in range(ncached):
        x1_scr[t * _CTILE : (t + 1) * _CTILE, :] = jnp.dot(
            ltop_scr[t * _CTILE : (t + 1) * _CTILE, :],
            x0_scr[...],
            preferred_element_type=jnp.float32,
        )

    def body(i, carry):
        slot = i % _NBUF
        wait_chunk(i)

        @pl.when(i + _NBUF - 1 < total)
        def _issue_next():
            start_chunk(i + _NBUF - 1)

        row = _C + (i % nstream) * _CH

        @pl.when(i < nstream)
        def _first_pass_chunk():
            x1_scr[pl.ds(row, _CH), :] = jnp.dot(
                bufs[slot], x0_scr[...], preferred_element_type=jnp.float32
            )

        @pl.when(i >= nstream)
        def _second_pass_chunk():
            second_pass_tile(bufs[slot], row, _CH)

        for t in range(ncached):
            @pl.when(i == nstream + t * (nstream // ncached))
            def _second_pass_cached():
                second_pass_tile(
                    ltop_scr[t * _CTILE : (t + 1) * _CTILE, :], t * _CTILE, _CTILE
                )

        return carry

    jax.lax.fori_loop(0, total, body, 0)


def kernel(laplacian, inputs, weight, bias, precompute=0, einsum=0):
    B, Fin, V, X, Y, Z = inputs.shape
    K, _, Fout = weight.shape
    F = Fin * B * X * Y * Z

    x0t = inputs.reshape(F, V)
    w3 = weight.reshape(K * Fin, Fout)
    b2d = bias.reshape(Fout, 1)

    out_t = pl.pallas_call(
        _cheb_kernel,
        in_specs=[
            pl.BlockSpec((F, V), lambda: (0, 0)),
            pl.BlockSpec((K * F, Fout), lambda: (0, 0)),
            pl.BlockSpec((Fout, 1), lambda: (0, 0)),
            pl.BlockSpec(memory_space=pl.ANY),
        ],
        out_specs=pl.BlockSpec((Fout, V), lambda: (0, 0)),
        out_shape=jax.ShapeDtypeStruct((Fout, V), jnp.float32),
        scratch_shapes=[
            pltpu.VMEM((_C, V), jnp.float32),
            pltpu.VMEM((V, F), jnp.float32),
            pltpu.VMEM((V, F), jnp.float32),
            pltpu.VMEM((_NBUF, _CH, V), jnp.float32),
            pltpu.SemaphoreType.DMA((_NBUF, 2)),
            pltpu.SemaphoreType.DMA((4,)),
        ],
    )(x0t, w3, b2d, laplacian)

    return out_t.reshape(B, Fout, V, X, Y, Z)


# speedup vs baseline: 1.0189x; 1.0189x over previous
import jax
import jax.numpy as jnp
from jax.experimental import pallas as pl
from jax.experimental.pallas import tpu as pltpu

_CH = 256
_NBUF = 4
_PTILE = 512


def _cheb_kernel(
    x0t_ref, w_ref, b_ref, l_hbm, out_ref, lc_scr, x0_scr, x1_scr, x1b_scr, bufs, sems
):
    f = x0t_ref.shape[0]
    v = x0t_ref.shape[1]
    nchunks = v // _CH

    def chunk_copy(i):
        return pltpu.make_async_copy(
            l_hbm.at[pl.ds(i * _CH, _CH), :],
            bufs.at[i % _NBUF],
            sems.at[i % _NBUF],
        )

    for i in range(_NBUF - 1):
        chunk_copy(i).start()

    x0_scr[...] = jnp.transpose(x0t_ref[...], (1, 0))

    def body(i, carry):
        slot = i % _NBUF
        chunk_copy(i).wait()

        @pl.when(i + _NBUF - 1 < nchunks)
        def _issue_next():
            chunk_copy(i + _NBUF - 1).start()

        l_chunk = bufs[slot]
        x1c = jnp.dot(l_chunk, x0_scr[...], preferred_element_type=jnp.float32)
        x1_scr[pl.ds(i * _CH, _CH), :] = x1c
        x1b_scr[pl.ds(i * _CH, _CH), :] = x1c.astype(jnp.bfloat16)
        lc_scr[pl.ds(i * _CH, _CH), :] = l_chunk.astype(jnp.bfloat16)
        return carry

    jax.lax.fori_loop(0, nchunks, body, 0)

    for t in range(v // _PTILE):
        row = t * _PTILE
        y = jnp.dot(
            lc_scr[row : row + _PTILE, :],
            x1b_scr[...],
            preferred_element_type=jnp.float32,
        )
        x0_r = x0_scr[row : row + _PTILE, :]
        x1_r = x1_scr[row : row + _PTILE, :]
        x2_r = 2.0 * y - x0_r
        acc = jnp.dot(x0_r, w_ref[0:f, :], preferred_element_type=jnp.float32)
        acc += jnp.dot(x1_r, w_ref[f : 2 * f, :], preferred_element_type=jnp.float32)
        acc += jnp.dot(x2_r, w_ref[2 * f : 3 * f, :], preferred_element_type=jnp.float32)
        out_ref[:, row : row + _PTILE] = jnp.transpose(acc, (1, 0)) + b_ref[...]


def kernel(laplacian, inputs, weight, bias, precompute=0, einsum=0):
    B, Fin, V, X, Y, Z = inputs.shape
    K, _, Fout = weight.shape
    F = Fin * B * X * Y * Z

    x0t = inputs.reshape(F, V)
    w3 = weight.reshape(K * Fin, Fout)
    b2d = bias.reshape(Fout, 1)

    out_t = pl.pallas_call(
        _cheb_kernel,
        in_specs=[
            pl.BlockSpec((F, V), lambda: (0, 0)),
            pl.BlockSpec((K * F, Fout), lambda: (0, 0)),
            pl.BlockSpec((Fout, 1), lambda: (0, 0)),
            pl.BlockSpec(memory_space=pl.ANY),
        ],
        out_specs=pl.BlockSpec((Fout, V), lambda: (0, 0)),
        out_shape=jax.ShapeDtypeStruct((Fout, V), jnp.float32),
        scratch_shapes=[
            pltpu.VMEM((V, V), jnp.bfloat16),
            pltpu.VMEM((V, F), jnp.float32),
            pltpu.VMEM((V, F), jnp.float32),
            pltpu.VMEM((V, F), jnp.bfloat16),
            pltpu.VMEM((_NBUF, _CH, V), jnp.float32),
            pltpu.SemaphoreType.DMA((_NBUF,)),
        ],
    )(x0t, w3, b2d, laplacian)

    return out_t.reshape(B, Fout, V, X, Y, Z)


# speedup vs baseline: 1.1385x; 1.1174x over previous
import jax
import jax.numpy as jnp
from jax.experimental import pallas as pl
from jax.experimental.pallas import tpu as pltpu

_TILE = 512


def _cheb_kernel(l_ref, x0t_ref, w_ref, b_ref, out_ref, lc_scr, x0_scr, x1_scr, x1b_scr):
    k = pl.program_id(0)
    r = pl.program_id(1)
    f = x0t_ref.shape[0]

    @pl.when(jnp.logical_and(k == 0, r == 0))
    def _transpose_x0():
        x0_scr[...] = jnp.transpose(x0t_ref[...], (1, 0))

    @pl.when(k == 0)
    def _first_pass():
        l_tile = l_ref[...]
        x1c = jnp.dot(l_tile, x0_scr[...], preferred_element_type=jnp.float32)
        x1_scr[pl.ds(r * _TILE, _TILE), :] = x1c
        x1b_scr[pl.ds(r * _TILE, _TILE), :] = x1c.astype(jnp.bfloat16)
        lc_scr[pl.ds(r * _TILE, _TILE), :] = l_tile.astype(jnp.bfloat16)

    @pl.when(k == 1)
    def _second_pass():
        row = r * _TILE
        y = jnp.dot(
            lc_scr[pl.ds(row, _TILE), :],
            x1b_scr[...],
            preferred_element_type=jnp.float32,
        )
        x0_r = x0_scr[pl.ds(row, _TILE), :]
        x1_r = x1_scr[pl.ds(row, _TILE), :]
        x2_r = 2.0 * y - x0_r
        acc = jnp.dot(x0_r, w_ref[0:f, :], preferred_element_type=jnp.float32)
        acc += jnp.dot(x1_r, w_ref[f : 2 * f, :], preferred_element_type=jnp.float32)
        acc += jnp.dot(x2_r, w_ref[2 * f : 3 * f, :], preferred_element_type=jnp.float32)
        out_ref[...] = jnp.transpose(acc, (1, 0)) + b_ref[...]


def kernel(laplacian, inputs, weight, bias, precompute=0, einsum=0):
    B, Fin, V, X, Y, Z = inputs.shape
    K, _, Fout = weight.shape
    F = Fin * B * X * Y * Z

    x0t = inputs.reshape(F, V)
    w3 = weight.reshape(K * Fin, Fout)
    b2d = bias.reshape(Fout, 1)

    R = V // _TILE

    out_t = pl.pallas_call(
        _cheb_kernel,
        grid=(2, R),
        in_specs=[
            pl.BlockSpec((_TILE, V), lambda k, r: (jnp.where(k == 0, r, R - 1), 0)),
            pl.BlockSpec((F, V), lambda k, r: (0, 0)),
            pl.BlockSpec((K * F, Fout), lambda k, r: (0, 0)),
            pl.BlockSpec((Fout, 1), lambda k, r: (0, 0)),
        ],
        out_specs=pl.BlockSpec((Fout, _TILE), lambda k, r: (0, jnp.where(k == 1, r, 0))),
        out_shape=jax.ShapeDtypeStruct((Fout, V), jnp.float32),
        scratch_shapes=[
            pltpu.VMEM((V, V), jnp.bfloat16),
            pltpu.VMEM((V, F), jnp.float32),
            pltpu.VMEM((V, F), jnp.float32),
            pltpu.VMEM((V, F), jnp.bfloat16),
        ],
    )(laplacian, x0t, w3, b2d)

    return out_t.reshape(B, Fout, V, X, Y, Z)
